# Initial kernel scaffold; baseline (speedup 1.0000x reference)
import functools

import jax
import jax.numpy as jnp
from jax import lax
from jax.experimental import pallas as pl
from jax.experimental.pallas import tpu as pltpu
from jax.experimental.pallas import tpu_sc as plsc

N_NODES = 10000
N_EDGES = 320000
D_IN = 128
HIDDEN = 64
NUM_CLASSES = 40
CPAD = 48

NC = 2
NS = 16
NW = NC * NS
BLK = 128
NBLK = 80
EPAD = NW * NBLK * BLK
ACC_ROWS = 10240
STRIPE = ACC_ROWS // NS
TRASH = N_NODES + 16

_mesh = plsc.VectorSubcoreMesh(core_axis_name="c", subcore_axis_name="s")


def _sc_degree(dst_r):
  @functools.partial(
      pl.kernel,
      mesh=_mesh,
      out_type=jax.ShapeDtypeStruct((NC, ACC_ROWS, 16), jnp.float32),
      scratch_types=[
          pltpu.VMEM((NBLK, BLK), jnp.int32),
          pltpu.VMEM((BLK, 16), jnp.float32),
          pltpu.VMEM((BLK, 16), jnp.float32),
          pltpu.VMEM_SHARED((ACC_ROWS, 16), jnp.float32),
      ],
  )
  def k(dst_hbm, out_hbm, dst_v, ones_v, zeros_v, acc):
    c = lax.axis_index("c")
    s = lax.axis_index("s")
    wid = c * NS + s
    pltpu.sync_copy(dst_hbm.at[wid], dst_v)

    def fill(i, _):
      ones_v[i, :] = jnp.full((16,), 1.0, jnp.float32)
      zeros_v[i, :] = jnp.zeros((16,), jnp.float32)
      return 0

    lax.fori_loop(0, BLK, fill, 0)
    base = s * STRIPE
    for t in range(STRIPE // BLK):
      pltpu.sync_copy(zeros_v, acc.at[pl.ds(base + t * BLK, BLK)])
    plsc.subcore_barrier()

    def blk(j, _):
      pltpu.sync_copy(ones_v, acc.at[dst_v.at[j]], add=True)
      return 0

    lax.fori_loop(0, NBLK, blk, 0)
    plsc.subcore_barrier()
    for t in range(STRIPE // BLK):
      r = base + t * BLK
      pltpu.sync_copy(acc.at[pl.ds(r, BLK)], out_hbm.at[c, pl.ds(r, BLK)])

  return k(dst_r)


def _sc_aggregate(src_r, dst_r, feat, width):
  @functools.partial(
      pl.kernel,
      mesh=_mesh,
      out_type=jax.ShapeDtypeStruct((NC, ACC_ROWS, width), jnp.float32),
      scratch_types=[
          pltpu.VMEM((NBLK, BLK), jnp.int32),
          pltpu.VMEM((NBLK, BLK), jnp.int32),
          pltpu.VMEM((BLK, width), jnp.float32),
          pltpu.VMEM_SHARED((ACC_ROWS, width), jnp.float32),
      ],
  )
  def k(src_hbm, dst_hbm, feat_hbm, out_hbm, src_v, dst_v, buf, acc):
    c = lax.axis_index("c")
    s = lax.axis_index("s")
    wid = c * NS + s
    pltpu.sync_copy(src_hbm.at[wid], src_v)
    pltpu.sync_copy(dst_hbm.at[wid], dst_v)

    def fill(i, _):
      for t in range(width // 16):
        buf[i, pl.ds(t * 16, 16)] = jnp.zeros((16,), jnp.float32)
      return 0

    lax.fori_loop(0, BLK, fill, 0)
    base = s * STRIPE
    for t in range(STRIPE // BLK):
      pltpu.sync_copy(buf, acc.at[pl.ds(base + t * BLK, BLK)])
    plsc.subcore_barrier()

    def blk(j, _):
      pltpu.sync_copy(feat_hbm.at[src_v.at[j]], buf)
      pltpu.sync_copy(buf, acc.at[dst_v.at[j]], add=True)
      return 0

    lax.fori_loop(0, NBLK, blk, 0)
    plsc.subcore_barrier()
    for t in range(STRIPE // BLK):
      r = base + t * BLK
      pltpu.sync_copy(acc.at[pl.ds(r, BLK)], out_hbm.at[c, pl.ds(r, BLK)])

  return k(src_r, dst_r, feat)


_ROWS = 1000
_GRID = N_NODES // _ROWS


def _dinv_of(dp_ref):
  deg = dp_ref[0, :, 0:1] + dp_ref[1, :, 0:1] + 1.0
  return lax.rsqrt(deg)


def _l1_body(x_ref, w_ref, dp_ref, y_ref):
  dinv = _dinv_of(dp_ref)
  y_ref[...] = jnp.dot(
      x_ref[...], w_ref[...], preferred_element_type=jnp.float32) * dinv


def _l2_body(y_ref, p_ref, dp_ref, b1_ref, w2_ref, z_ref):
  dinv = _dinv_of(dp_ref)
  pre = (p_ref[0] + p_ref[1] + y_ref[...]) * dinv + b1_ref[...]
  h = jnp.maximum(pre, 0.0)
  z_ref[...] = jnp.dot(
      h, w2_ref[...], preferred_element_type=jnp.float32) * dinv


def _fin_body(z_ref, q_ref, dp_ref, b2_ref, o_ref):
  dinv = _dinv_of(dp_ref)
  o = (q_ref[0] + q_ref[1] + z_ref[...]) * dinv + b2_ref[...]
  col = lax.broadcasted_iota(jnp.int32, o.shape, 1)
  valid = col < NUM_CLASSES
  om = jnp.where(valid, o, -jnp.inf)
  m = jnp.max(om, axis=1, keepdims=True)
  e = jnp.where(valid, jnp.exp(om - m), 0.0)
  lse = jnp.log(jnp.sum(e, axis=1, keepdims=True)) + m
  o_ref[...] = (o - lse)[:, :NUM_CLASSES]


def _dp_spec():
  return pl.BlockSpec((NC, _ROWS, 16), lambda i: (0, i, 0))


def kernel(x, edge_index, W1, b1, W2, b2):
  src = edge_index[0].astype(jnp.int32)
  dst = edge_index[1].astype(jnp.int32)
  pad = EPAD - N_EDGES
  src_r = jnp.concatenate([src, jnp.zeros((pad,), jnp.int32)]).reshape(
      NW, NBLK, BLK)
  dst_r = jnp.concatenate([dst, jnp.full((pad,), TRASH, jnp.int32)]).reshape(
      NW, NBLK, BLK)
  w2p = jnp.pad(W2, ((0, 0), (0, CPAD - NUM_CLASSES)))
  b1r = b1.reshape(1, HIDDEN)
  b2r = jnp.pad(b2, (0, CPAD - NUM_CLASSES)).reshape(1, CPAD)

  degp = _sc_degree(dst_r)

  y = pl.pallas_call(
      _l1_body,
      grid=(_GRID,),
      in_specs=[
          pl.BlockSpec((_ROWS, D_IN), lambda i: (i, 0)),
          pl.BlockSpec((D_IN, HIDDEN), lambda i: (0, 0)),
          _dp_spec(),
      ],
      out_specs=pl.BlockSpec((_ROWS, HIDDEN), lambda i: (i, 0)),
      out_shape=jax.ShapeDtypeStruct((N_NODES, HIDDEN), jnp.float32),
  )(x, W1, degp)

  p = _sc_aggregate(src_r, dst_r, y, HIDDEN)

  z = pl.pallas_call(
      _l2_body,
      grid=(_GRID,),
      in_specs=[
          pl.BlockSpec((_ROWS, HIDDEN), lambda i: (i, 0)),
          pl.BlockSpec((NC, _ROWS, HIDDEN), lambda i: (0, i, 0)),
          _dp_spec(),
          pl.BlockSpec((1, HIDDEN), lambda i: (0, 0)),
          pl.BlockSpec((HIDDEN, CPAD), lambda i: (0, 0)),
      ],
      out_specs=pl.BlockSpec((_ROWS, CPAD), lambda i: (i, 0)),
      out_shape=jax.ShapeDtypeStruct((N_NODES, CPAD), jnp.float32),
  )(y, p, degp, b1r, w2p)

  q = _sc_aggregate(src_r, dst_r, z, CPAD)

  out = pl.pallas_call(
      _fin_body,
      grid=(_GRID,),
      in_specs=[
          pl.BlockSpec((_ROWS, CPAD), lambda i: (i, 0)),
          pl.BlockSpec((NC, _ROWS, CPAD), lambda i: (0, i, 0)),
          _dp_spec(),
          pl.BlockSpec((1, CPAD), lambda i: (0, 0)),
      ],
      out_specs=pl.BlockSpec((_ROWS, NUM_CLASSES), lambda i: (i, 0)),
      out_shape=jax.ShapeDtypeStruct((N_NODES, NUM_CLASSES), jnp.float32),
  )(z, q, degp, b2r)

  return out


# speedup vs baseline: 15.2751x; 15.2751x over previous
import functools

import jax
import jax.numpy as jnp
from jax import lax
from jax.experimental import pallas as pl
from jax.experimental.pallas import tpu as pltpu
from jax.experimental.pallas import tpu_sc as plsc

N_NODES = 10000
N_EDGES = 320000
D_IN = 128
HIDDEN = 64
NUM_CLASSES = 40
CPAD = 48

NC = 2
NS = 16
NW = NC * NS
BLK = 128
NBLK = 80
EPAD = NW * NBLK * BLK
ACC_ROWS = 10240
STRIPE = ACC_ROWS // NS
TRASH = N_NODES + 16

_mesh = plsc.VectorSubcoreMesh(core_axis_name="c", subcore_axis_name="s")
_sc_params = pltpu.CompilerParams(use_tc_tiling_on_sc=False)


def _sc_degree(dst_r):
  @functools.partial(
      pl.kernel,
      mesh=_mesh,
      out_type=jax.ShapeDtypeStruct((NC, ACC_ROWS, 16), jnp.float32),
      compiler_params=_sc_params,
      scratch_types=[
          pltpu.VMEM((NBLK, BLK), jnp.int32),
          pltpu.VMEM((BLK, 16), jnp.float32),
          pltpu.VMEM((BLK, 16), jnp.float32),
          pltpu.VMEM_SHARED((ACC_ROWS, 16), jnp.float32),
      ],
  )
  def k(dst_hbm, out_hbm, dst_v, ones_v, zeros_v, acc):
    c = lax.axis_index("c")
    s = lax.axis_index("s")
    wid = c * NS + s
    pltpu.sync_copy(dst_hbm.at[wid], dst_v)

    def fill(i, _):
      ones_v[i, :] = jnp.full((16,), 1.0, jnp.float32)
      zeros_v[i, :] = jnp.zeros((16,), jnp.float32)
      return 0

    lax.fori_loop(0, BLK, fill, 0)
    base = s * STRIPE
    for t in range(STRIPE // BLK):
      pltpu.sync_copy(zeros_v, acc.at[pl.ds(base + t * BLK, BLK)])
    plsc.subcore_barrier()

    def blk(j, _):
      pltpu.sync_copy(ones_v, acc.at[dst_v.at[j]], add=True)
      return 0

    lax.fori_loop(0, NBLK, blk, 0)
    plsc.subcore_barrier()
    for t in range(STRIPE // BLK):
      r = base + t * BLK
      pltpu.sync_copy(acc.at[pl.ds(r, BLK)], out_hbm.at[c, pl.ds(r, BLK)])

  return k(dst_r)


def _sc_aggregate(src_r, dst_r, feat, width):
  @functools.partial(
      pl.kernel,
      mesh=_mesh,
      out_type=jax.ShapeDtypeStruct((NC, ACC_ROWS, width), jnp.float32),
      compiler_params=_sc_params,
      scratch_types=[
          pltpu.VMEM((NBLK, BLK), jnp.int32),
          pltpu.VMEM((NBLK, BLK), jnp.int32),
          pltpu.VMEM((BLK, width), jnp.float32),
          pltpu.VMEM_SHARED((ACC_ROWS, width), jnp.float32),
      ],
  )
  def k(src_hbm, dst_hbm, feat_hbm, out_hbm, src_v, dst_v, buf, acc):
    c = lax.axis_index("c")
    s = lax.axis_index("s")
    wid = c * NS + s
    pltpu.sync_copy(src_hbm.at[wid], src_v)
    pltpu.sync_copy(dst_hbm.at[wid], dst_v)

    def fill(i, _):
      for t in range(width // 16):
        buf[i, pl.ds(t * 16, 16)] = jnp.zeros((16,), jnp.float32)
      return 0

    lax.fori_loop(0, BLK, fill, 0)
    base = s * STRIPE
    for t in range(STRIPE // BLK):
      pltpu.sync_copy(buf, acc.at[pl.ds(base + t * BLK, BLK)])
    plsc.subcore_barrier()

    def blk(j, _):
      pltpu.sync_copy(feat_hbm.at[src_v.at[j]], buf)
      pltpu.sync_copy(buf, acc.at[dst_v.at[j]], add=True)
      return 0

    lax.fori_loop(0, NBLK, blk, 0)
    plsc.subcore_barrier()
    for t in range(STRIPE // BLK):
      r = base + t * BLK
      pltpu.sync_copy(acc.at[pl.ds(r, BLK)], out_hbm.at[c, pl.ds(r, BLK)])

  return k(src_r, dst_r, feat)


_ROWS = 1000
_GRID = N_NODES // _ROWS


def _dinv_of(dp_ref):
  deg = dp_ref[0, :, 0:1] + dp_ref[1, :, 0:1] + 1.0
  return lax.rsqrt(deg)


def _l1_body(x_ref, w_ref, dp_ref, y_ref):
  dinv = _dinv_of(dp_ref)
  y_ref[...] = jnp.dot(
      x_ref[...], w_ref[...], preferred_element_type=jnp.float32) * dinv


def _l2_body(y_ref, p_ref, dp_ref, b1_ref, w2_ref, z_ref):
  dinv = _dinv_of(dp_ref)
  pre = (p_ref[0] + p_ref[1] + y_ref[...]) * dinv + b1_ref[...]
  h = jnp.maximum(pre, 0.0)
  z_ref[...] = jnp.dot(
      h, w2_ref[...], preferred_element_type=jnp.float32) * dinv


def _fin_body(z_ref, q_ref, dp_ref, b2_ref, o_ref):
  dinv = _dinv_of(dp_ref)
  o = (q_ref[0] + q_ref[1] + z_ref[...]) * dinv + b2_ref[...]
  col = lax.broadcasted_iota(jnp.int32, o.shape, 1)
  valid = col < NUM_CLASSES
  om = jnp.where(valid, o, -jnp.inf)
  m = jnp.max(om, axis=1, keepdims=True)
  e = jnp.where(valid, jnp.exp(om - m), 0.0)
  lse = jnp.log(jnp.sum(e, axis=1, keepdims=True)) + m
  o_ref[...] = (o - lse)[:, :NUM_CLASSES]


def _dp_spec():
  return pl.BlockSpec((NC, _ROWS, 16), lambda i: (0, i, 0))


def kernel(x, edge_index, W1, b1, W2, b2):
  src = edge_index[0].astype(jnp.int32)
  dst = edge_index[1].astype(jnp.int32)
  pad = EPAD - N_EDGES
  src_r = jnp.concatenate([src, jnp.zeros((pad,), jnp.int32)]).reshape(
      NW, NBLK, BLK)
  dst_r = jnp.concatenate([dst, jnp.full((pad,), TRASH, jnp.int32)]).reshape(
      NW, NBLK, BLK)
  w2p = jnp.pad(W2, ((0, 0), (0, CPAD - NUM_CLASSES)))
  b1r = b1.reshape(1, HIDDEN)
  b2r = jnp.pad(b2, (0, CPAD - NUM_CLASSES)).reshape(1, CPAD)

  degp = _sc_degree(dst_r)

  y = pl.pallas_call(
      _l1_body,
      grid=(_GRID,),
      in_specs=[
          pl.BlockSpec((_ROWS, D_IN), lambda i: (i, 0)),
          pl.BlockSpec((D_IN, HIDDEN), lambda i: (0, 0)),
          _dp_spec(),
      ],
      out_specs=pl.BlockSpec((_ROWS, HIDDEN), lambda i: (i, 0)),
      out_shape=jax.ShapeDtypeStruct((N_NODES, HIDDEN), jnp.float32),
  )(x, W1, degp)

  p = _sc_aggregate(src_r, dst_r, y, HIDDEN)

  z = pl.pallas_call(
      _l2_body,
      grid=(_GRID,),
      in_specs=[
          pl.BlockSpec((_ROWS, HIDDEN), lambda i: (i, 0)),
          pl.BlockSpec((NC, _ROWS, HIDDEN), lambda i: (0, i, 0)),
          _dp_spec(),
          pl.BlockSpec((1, HIDDEN), lambda i: (0, 0)),
          pl.BlockSpec((HIDDEN, CPAD), lambda i: (0, 0)),
      ],
      out_specs=pl.BlockSpec((_ROWS, CPAD), lambda i: (i, 0)),
      out_shape=jax.ShapeDtypeStruct((N_NODES, CPAD), jnp.float32),
  )(y, p, degp, b1r, w2p)

  q = _sc_aggregate(src_r, dst_r, z, CPAD)

  out = pl.pallas_call(
      _fin_body,
      grid=(_GRID,),
      in_specs=[
          pl.BlockSpec((_ROWS, CPAD), lambda i: (i, 0)),
          pl.BlockSpec((NC, _ROWS, CPAD), lambda i: (0, i, 0)),
          _dp_spec(),
          pl.BlockSpec((1, CPAD), lambda i: (0, 0)),
      ],
      out_specs=pl.BlockSpec((_ROWS, NUM_CLASSES), lambda i: (i, 0)),
      out_shape=jax.ShapeDtypeStruct((N_NODES, NUM_CLASSES), jnp.float32),
  )(z, q, degp, b2r)

  return out


# speedup vs baseline: 17.7729x; 1.1635x over previous
import functools

import jax
import jax.numpy as jnp
from jax import lax
from jax.experimental import pallas as pl
from jax.experimental.pallas import tpu as pltpu
from jax.experimental.pallas import tpu_sc as plsc

N_NODES = 10000
N_EDGES = 320000
D_IN = 128
HIDDEN = 64
NUM_CLASSES = 40
CPAD = 48

NC = 2
NS = 16
NW = NC * NS
BLK = 128
NBLK = 80
EPAD = NW * NBLK * BLK
ACC_ROWS = 10240
STRIPE = ACC_ROWS // NS
TRASH = N_NODES + 16

_mesh = plsc.VectorSubcoreMesh(core_axis_name="c", subcore_axis_name="s")
_sc_params = pltpu.CompilerParams(use_tc_tiling_on_sc=False)


def _sc_degree(dst_r):
  @functools.partial(
      pl.kernel,
      mesh=_mesh,
      out_type=jax.ShapeDtypeStruct((NC, ACC_ROWS, 16), jnp.float32),
      compiler_params=_sc_params,
      scratch_types=[
          pltpu.VMEM((NBLK, BLK), jnp.int32),
          pltpu.VMEM((BLK, 16), jnp.float32),
          pltpu.VMEM((BLK, 16), jnp.float32),
          pltpu.VMEM_SHARED((ACC_ROWS, 16), jnp.float32),
      ],
  )
  def k(dst_hbm, out_hbm, dst_v, ones_v, zeros_v, acc):
    c = lax.axis_index("c")
    s = lax.axis_index("s")
    wid = c * NS + s
    pltpu.sync_copy(dst_hbm.at[wid], dst_v)

    def fill(i, _):
      ones_v[i, :] = jnp.full((16,), 1.0, jnp.float32)
      zeros_v[i, :] = jnp.zeros((16,), jnp.float32)
      return 0

    lax.fori_loop(0, BLK, fill, 0)
    base = s * STRIPE
    for t in range(STRIPE // BLK):
      pltpu.sync_copy(zeros_v, acc.at[pl.ds(base + t * BLK, BLK)])
    plsc.subcore_barrier()

    def blk(j, _):
      pltpu.sync_copy(ones_v, acc.at[dst_v.at[j]], add=True)
      return 0

    lax.fori_loop(0, NBLK, blk, 0)
    plsc.subcore_barrier()
    for t in range(STRIPE // BLK):
      r = base + t * BLK
      pltpu.sync_copy(acc.at[pl.ds(r, BLK)], out_hbm.at[c, pl.ds(r, BLK)])

  return k(dst_r)


def _sc_aggregate(src_r, dst_r, feat, width):
  @functools.partial(
      pl.kernel,
      mesh=_mesh,
      out_type=jax.ShapeDtypeStruct((NC, ACC_ROWS, width), jnp.float32),
      compiler_params=_sc_params,
      scratch_types=[
          pltpu.VMEM((NBLK, BLK), jnp.int32),
          pltpu.VMEM((NBLK, BLK), jnp.int32),
          pltpu.VMEM((BLK, width), jnp.float32),
          pltpu.VMEM((BLK, width), jnp.float32),
          pltpu.VMEM_SHARED((ACC_ROWS, width), jnp.float32),
          pltpu.SemaphoreType.DMA,
          pltpu.SemaphoreType.DMA,
      ],
  )
  def k(src_hbm, dst_hbm, feat_hbm, out_hbm, src_v, dst_v, buf0, buf1, acc,
        sem0, sem1):
    c = lax.axis_index("c")
    s = lax.axis_index("s")
    wid = c * NS + s
    pltpu.sync_copy(src_hbm.at[wid], src_v)
    pltpu.sync_copy(dst_hbm.at[wid], dst_v)

    def fill(i, _):
      for t in range(width // 16):
        buf0[i, pl.ds(t * 16, 16)] = jnp.zeros((16,), jnp.float32)
      return 0

    lax.fori_loop(0, BLK, fill, 0)
    base = s * STRIPE
    for t in range(STRIPE // BLK):
      pltpu.sync_copy(buf0, acc.at[pl.ds(base + t * BLK, BLK)])
    plsc.subcore_barrier()

    pltpu.async_copy(feat_hbm.at[src_v.at[0]], buf0, sem0)

    def blk(t, _):
      j0 = 2 * t
      pltpu.async_copy(feat_hbm.at[src_v.at[j0 + 1]], buf1, sem1)
      pltpu.make_async_copy(feat_hbm.at[src_v.at[j0]], buf0, sem0).wait()
      pltpu.sync_copy(buf0, acc.at[dst_v.at[j0]], add=True)

      @pl.when(t < NBLK // 2 - 1)
      def _():
        pltpu.async_copy(feat_hbm.at[src_v.at[j0 + 2]], buf0, sem0)

      pltpu.make_async_copy(feat_hbm.at[src_v.at[j0 + 1]], buf1, sem1).wait()
      pltpu.sync_copy(buf1, acc.at[dst_v.at[j0 + 1]], add=True)
      return 0

    lax.fori_loop(0, NBLK // 2, blk, 0)
    plsc.subcore_barrier()
    for t in range(STRIPE // BLK):
      r = base + t * BLK
      pltpu.sync_copy(acc.at[pl.ds(r, BLK)], out_hbm.at[c, pl.ds(r, BLK)])

  return k(src_r, dst_r, feat)


_ROWS = 1000
_GRID = N_NODES // _ROWS


def _dinv_of(dp_ref):
  deg = dp_ref[0, :, 0:1] + dp_ref[1, :, 0:1] + 1.0
  return lax.rsqrt(deg)


def _l1_body(x_ref, w_ref, dp_ref, y_ref):
  dinv = _dinv_of(dp_ref)
  y_ref[...] = jnp.dot(
      x_ref[...], w_ref[...], preferred_element_type=jnp.float32) * dinv


def _l2_body(y_ref, p_ref, dp_ref, b1_ref, w2_ref, z_ref):
  dinv = _dinv_of(dp_ref)
  pre = (p_ref[0] + p_ref[1] + y_ref[...]) * dinv + b1_ref[...]
  h = jnp.maximum(pre, 0.0)
  z_ref[...] = jnp.dot(
      h, w2_ref[...], preferred_element_type=jnp.float32) * dinv


def _fin_body(z_ref, q_ref, dp_ref, b2_ref, o_ref):
  dinv = _dinv_of(dp_ref)
  o = (q_ref[0] + q_ref[1] + z_ref[...]) * dinv + b2_ref[...]
  col = lax.broadcasted_iota(jnp.int32, o.shape, 1)
  valid = col < NUM_CLASSES
  om = jnp.where(valid, o, -jnp.inf)
  m = jnp.max(om, axis=1, keepdims=True)
  e = jnp.where(valid, jnp.exp(om - m), 0.0)
  lse = jnp.log(jnp.sum(e, axis=1, keepdims=True)) + m
  o_ref[...] = (o - lse)[:, :NUM_CLASSES]


def _dp_spec():
  return pl.BlockSpec((NC, _ROWS, 16), lambda i: (0, i, 0))


def kernel(x, edge_index, W1, b1, W2, b2):
  src = edge_index[0].astype(jnp.int32)
  dst = edge_index[1].astype(jnp.int32)
  pad = EPAD - N_EDGES
  src_r = jnp.concatenate([src, jnp.zeros((pad,), jnp.int32)]).reshape(
      NW, NBLK, BLK)
  dst_r = jnp.concatenate([dst, jnp.full((pad,), TRASH, jnp.int32)]).reshape(
      NW, NBLK, BLK)
  w2p = jnp.pad(W2, ((0, 0), (0, CPAD - NUM_CLASSES)))
  b1r = b1.reshape(1, HIDDEN)
  b2r = jnp.pad(b2, (0, CPAD - NUM_CLASSES)).reshape(1, CPAD)

  degp = _sc_degree(dst_r)

  y = pl.pallas_call(
      _l1_body,
      grid=(_GRID,),
      in_specs=[
          pl.BlockSpec((_ROWS, D_IN), lambda i: (i, 0)),
          pl.BlockSpec((D_IN, HIDDEN), lambda i: (0, 0)),
          _dp_spec(),
      ],
      out_specs=pl.BlockSpec((_ROWS, HIDDEN), lambda i: (i, 0)),
      out_shape=jax.ShapeDtypeStruct((N_NODES, HIDDEN), jnp.float32),
  )(x, W1, degp)

  p = _sc_aggregate(src_r, dst_r, y, HIDDEN)

  z = pl.pallas_call(
      _l2_body,
      grid=(_GRID,),
      in_specs=[
          pl.BlockSpec((_ROWS, HIDDEN), lambda i: (i, 0)),
          pl.BlockSpec((NC, _ROWS, HIDDEN), lambda i: (0, i, 0)),
          _dp_spec(),
          pl.BlockSpec((1, HIDDEN), lambda i: (0, 0)),
          pl.BlockSpec((HIDDEN, CPAD), lambda i: (0, 0)),
      ],
      out_specs=pl.BlockSpec((_ROWS, CPAD), lambda i: (i, 0)),
      out_shape=jax.ShapeDtypeStruct((N_NODES, CPAD), jnp.float32),
  )(y, p, degp, b1r, w2p)

  q = _sc_aggregate(src_r, dst_r, z, CPAD)

  out = pl.pallas_call(
      _fin_body,
      grid=(_GRID,),
      in_specs=[
          pl.BlockSpec((_ROWS, CPAD), lambda i: (i, 0)),
          pl.BlockSpec((NC, _ROWS, CPAD), lambda i: (0, i, 0)),
          _dp_spec(),
          pl.BlockSpec((1, CPAD), lambda i: (0, 0)),
      ],
      out_specs=pl.BlockSpec((_ROWS, NUM_CLASSES), lambda i: (i, 0)),
      out_shape=jax.ShapeDtypeStruct((N_NODES, NUM_CLASSES), jnp.float32),
  )(z, q, degp, b2r)

  return out


# speedup vs baseline: 35.6951x; 2.0084x over previous
import functools

import jax
import jax.numpy as jnp
from jax import lax
from jax.experimental import pallas as pl
from jax.experimental.pallas import tpu as pltpu
from jax.experimental.pallas import tpu_sc as plsc

N_NODES = 10000
N_EDGES = 320000
D_IN = 128
HIDDEN = 64
NUM_CLASSES = 40
CPAD = 48

NC = 2
NS = 16
NW = NC * NS
BLK = 128
NBLK = 80
EPAD = NW * NBLK * BLK
ACC_ROWS = 10240
STRIPE = ACC_ROWS // NS
TRASH = N_NODES + 16

_mesh = plsc.VectorSubcoreMesh(core_axis_name="c", subcore_axis_name="s")
_sc_params = pltpu.CompilerParams(use_tc_tiling_on_sc=False)


def _sc_degree(dst_r):
  @functools.partial(
      pl.kernel,
      mesh=_mesh,
      out_type=jax.ShapeDtypeStruct((NC, ACC_ROWS, 16), jnp.float32),
      compiler_params=_sc_params,
      scratch_types=[
          pltpu.VMEM((NBLK, BLK), jnp.int32),
          pltpu.VMEM((BLK, 16), jnp.float32),
          pltpu.VMEM((BLK, 16), jnp.float32),
          pltpu.VMEM_SHARED((ACC_ROWS, 16), jnp.float32),
      ],
  )
  def k(dst_hbm, out_hbm, dst_v, ones_v, zeros_v, acc):
    c = lax.axis_index("c")
    s = lax.axis_index("s")
    wid = c * NS + s
    pltpu.sync_copy(dst_hbm.at[wid], dst_v)

    def fill(i, _):
      ones_v[i, :] = jnp.full((16,), 1.0, jnp.float32)
      zeros_v[i, :] = jnp.zeros((16,), jnp.float32)
      return 0

    lax.fori_loop(0, BLK, fill, 0)
    base = s * STRIPE
    for t in range(STRIPE // BLK):
      pltpu.sync_copy(zeros_v, acc.at[pl.ds(base + t * BLK, BLK)])
    plsc.subcore_barrier()

    def blk(j, _):
      pltpu.sync_copy(ones_v, acc.at[dst_v.at[j]], add=True)
      return 0

    lax.fori_loop(0, NBLK, blk, 0)
    plsc.subcore_barrier()
    for t in range(STRIPE // BLK):
      r = base + t * BLK
      pltpu.sync_copy(acc.at[pl.ds(r, BLK)], out_hbm.at[c, pl.ds(r, BLK)])

  return k(dst_r)


def _sc_aggregate(src_r, dst_r, feat, width):
  @functools.partial(
      pl.kernel,
      mesh=_mesh,
      out_type=jax.ShapeDtypeStruct((NC, ACC_ROWS, width), jnp.float32),
      compiler_params=_sc_params,
      scratch_types=[
          pltpu.VMEM((NBLK, BLK), jnp.int32),
          pltpu.VMEM((NBLK, BLK), jnp.int32),
          pltpu.VMEM((BLK, width), jnp.float32),
          pltpu.VMEM((BLK, width), jnp.float32),
          pltpu.VMEM_SHARED((N_NODES, width), jnp.float32),
          pltpu.VMEM_SHARED((ACC_ROWS, width), jnp.float32),
          pltpu.SemaphoreType.DMA,
          pltpu.SemaphoreType.DMA,
      ],
  )
  def k(src_hbm, dst_hbm, feat_hbm, out_hbm, src_v, dst_v, buf0, buf1,
        feat_sh, acc, sem0, sem1):
    c = lax.axis_index("c")
    s = lax.axis_index("s")
    wid = c * NS + s
    pltpu.sync_copy(src_hbm.at[wid], src_v)
    pltpu.sync_copy(dst_hbm.at[wid], dst_v)

    frows = N_NODES // NS
    pltpu.sync_copy(feat_hbm.at[pl.ds(s * frows, frows)],
                    feat_sh.at[pl.ds(s * frows, frows)])

    def fill(i, _):
      for t in range(width // 16):
        buf0[i, pl.ds(t * 16, 16)] = jnp.zeros((16,), jnp.float32)
      return 0

    lax.fori_loop(0, BLK, fill, 0)
    base = s * STRIPE
    for t in range(STRIPE // BLK):
      pltpu.sync_copy(buf0, acc.at[pl.ds(base + t * BLK, BLK)])
    plsc.subcore_barrier()

    pltpu.async_copy(feat_sh.at[src_v.at[0]], buf0, sem0)

    def blk(t, _):
      j0 = 2 * t
      pltpu.async_copy(feat_sh.at[src_v.at[j0 + 1]], buf1, sem1)
      pltpu.make_async_copy(feat_sh.at[src_v.at[j0]], buf0, sem0).wait()
      pltpu.sync_copy(buf0, acc.at[dst_v.at[j0]], add=True)

      @pl.when(t < NBLK // 2 - 1)
      def _():
        pltpu.async_copy(feat_sh.at[src_v.at[j0 + 2]], buf0, sem0)

      pltpu.make_async_copy(feat_sh.at[src_v.at[j0 + 1]], buf1, sem1).wait()
      pltpu.sync_copy(buf1, acc.at[dst_v.at[j0 + 1]], add=True)
      return 0

    lax.fori_loop(0, NBLK // 2, blk, 0)
    plsc.subcore_barrier()
    for t in range(STRIPE // BLK):
      r = base + t * BLK
      pltpu.sync_copy(acc.at[pl.ds(r, BLK)], out_hbm.at[c, pl.ds(r, BLK)])

  return k(src_r, dst_r, feat)


_ROWS = 1000
_GRID = N_NODES // _ROWS


def _dinv_of(dp_ref):
  deg = dp_ref[0, :, 0:1] + dp_ref[1, :, 0:1] + 1.0
  return lax.rsqrt(deg)


def _l1_body(x_ref, w_ref, dp_ref, y_ref):
  dinv = _dinv_of(dp_ref)
  y_ref[...] = jnp.dot(
      x_ref[...], w_ref[...], preferred_element_type=jnp.float32) * dinv


def _l2_body(y_ref, p_ref, dp_ref, b1_ref, w2_ref, z_ref):
  dinv = _dinv_of(dp_ref)
  pre = (p_ref[0] + p_ref[1] + y_ref[...]) * dinv + b1_ref[...]
  h = jnp.maximum(pre, 0.0)
  z_ref[...] = jnp.dot(
      h, w2_ref[...], preferred_element_type=jnp.float32) * dinv


def _fin_body(z_ref, q_ref, dp_ref, b2_ref, o_ref):
  dinv = _dinv_of(dp_ref)
  o = (q_ref[0] + q_ref[1] + z_ref[...]) * dinv + b2_ref[...]
  col = lax.broadcasted_iota(jnp.int32, o.shape, 1)
  valid = col < NUM_CLASSES
  om = jnp.where(valid, o, -jnp.inf)
  m = jnp.max(om, axis=1, keepdims=True)
  e = jnp.where(valid, jnp.exp(om - m), 0.0)
  lse = jnp.log(jnp.sum(e, axis=1, keepdims=True)) + m
  o_ref[...] = (o - lse)[:, :NUM_CLASSES]


def _dp_spec():
  return pl.BlockSpec((NC, _ROWS, 16), lambda i: (0, i, 0))


def kernel(x, edge_index, W1, b1, W2, b2):
  src = edge_index[0].astype(jnp.int32)
  dst = edge_index[1].astype(jnp.int32)
  pad = EPAD - N_EDGES
  src_r = jnp.concatenate([src, jnp.zeros((pad,), jnp.int32)]).reshape(
      NW, NBLK, BLK)
  dst_r = jnp.concatenate([dst, jnp.full((pad,), TRASH, jnp.int32)]).reshape(
      NW, NBLK, BLK)
  w2p = jnp.pad(W2, ((0, 0), (0, CPAD - NUM_CLASSES)))
  b1r = b1.reshape(1, HIDDEN)
  b2r = jnp.pad(b2, (0, CPAD - NUM_CLASSES)).reshape(1, CPAD)

  degp = _sc_degree(dst_r)

  y = pl.pallas_call(
      _l1_body,
      grid=(_GRID,),
      in_specs=[
          pl.BlockSpec((_ROWS, D_IN), lambda i: (i, 0)),
          pl.BlockSpec((D_IN, HIDDEN), lambda i: (0, 0)),
          _dp_spec(),
      ],
      out_specs=pl.BlockSpec((_ROWS, HIDDEN), lambda i: (i, 0)),
      out_shape=jax.ShapeDtypeStruct((N_NODES, HIDDEN), jnp.float32),
  )(x, W1, degp)

  p = _sc_aggregate(src_r, dst_r, y, HIDDEN)

  z = pl.pallas_call(
      _l2_body,
      grid=(_GRID,),
      in_specs=[
          pl.BlockSpec((_ROWS, HIDDEN), lambda i: (i, 0)),
          pl.BlockSpec((NC, _ROWS, HIDDEN), lambda i: (0, i, 0)),
          _dp_spec(),
          pl.BlockSpec((1, HIDDEN), lambda i: (0, 0)),
          pl.BlockSpec((HIDDEN, CPAD), lambda i: (0, 0)),
      ],
      out_specs=pl.BlockSpec((_ROWS, CPAD), lambda i: (i, 0)),
      out_shape=jax.ShapeDtypeStruct((N_NODES, CPAD), jnp.float32),
  )(y, p, degp, b1r, w2p)

  q = _sc_aggregate(src_r, dst_r, z, CPAD)

  out = pl.pallas_call(
      _fin_body,
      grid=(_GRID,),
      in_specs=[
          pl.BlockSpec((_ROWS, CPAD), lambda i: (i, 0)),
          pl.BlockSpec((NC, _ROWS, CPAD), lambda i: (0, i, 0)),
          _dp_spec(),
          pl.BlockSpec((1, CPAD), lambda i: (0, 0)),
      ],
      out_specs=pl.BlockSpec((_ROWS, NUM_CLASSES), lambda i: (i, 0)),
      out_shape=jax.ShapeDtypeStruct((N_NODES, NUM_CLASSES), jnp.float32),
  )(z, q, degp, b2r)

  return out


# speedup vs baseline: 36.2148x; 1.0146x over previous
import functools

import jax
import jax.numpy as jnp
from jax import lax
from jax.experimental import pallas as pl
from jax.experimental.pallas import tpu as pltpu
from jax.experimental.pallas import tpu_sc as plsc

N_NODES = 10000
N_EDGES = 320000
D_IN = 128
HIDDEN = 64
NUM_CLASSES = 40
CPAD = 48

NC = 2
NS = 16
NW = NC * NS
BLK = 128
NBLK = 80
EPAD = NW * NBLK * BLK
ACC_ROWS = 10240
STRIPE = ACC_ROWS // NS
TRASH = N_NODES + 16

_mesh = plsc.VectorSubcoreMesh(core_axis_name="c", subcore_axis_name="s")
_sc_params = pltpu.CompilerParams(use_tc_tiling_on_sc=False)


def _sc_degree(dst_r):
  @functools.partial(
      pl.kernel,
      mesh=_mesh,
      out_type=jax.ShapeDtypeStruct((NC, ACC_ROWS, 16), jnp.float32),
      compiler_params=_sc_params,
      scratch_types=[
          pltpu.VMEM((NBLK, BLK), jnp.int32),
          pltpu.VMEM((BLK, 16), jnp.float32),
          pltpu.VMEM((BLK, 16), jnp.float32),
          pltpu.VMEM_SHARED((ACC_ROWS, 16), jnp.float32),
          pltpu.SemaphoreType.DMA,
      ],
  )
  def k(dst_hbm, out_hbm, dst_v, ones_v, zeros_v, acc, sem):
    c = lax.axis_index("c")
    s = lax.axis_index("s")
    wid = c * NS + s
    pltpu.sync_copy(dst_hbm.at[wid], dst_v)

    def fill(i, _):
      ones_v[i, :] = jnp.full((16,), 1.0, jnp.float32)
      zeros_v[i, :] = jnp.zeros((16,), jnp.float32)
      return 0

    lax.fori_loop(0, BLK, fill, 0)
    base = s * STRIPE
    for t in range(STRIPE // BLK):
      pltpu.sync_copy(zeros_v, acc.at[pl.ds(base + t * BLK, BLK)])
    plsc.subcore_barrier()

    def blk(j, _):
      pltpu.sync_copy(ones_v, acc.at[dst_v.at[j]], add=True)
      return 0

    lax.fori_loop(0, NBLK, blk, 0)
    plsc.subcore_barrier()
    for t in range(STRIPE // BLK):
      r = base + t * BLK
      pltpu.sync_copy(acc.at[pl.ds(r, BLK)], out_hbm.at[c, pl.ds(r, BLK)])

  return k(dst_r)


def _sc_aggregate(src_r, dst_r, feat, width):
  @functools.partial(
      pl.kernel,
      mesh=_mesh,
      out_type=jax.ShapeDtypeStruct((NC, ACC_ROWS, width), jnp.float32),
      compiler_params=_sc_params,
      scratch_types=[
          pltpu.VMEM((NBLK, BLK), jnp.int32),
          pltpu.VMEM((NBLK, BLK), jnp.int32),
          [pltpu.VMEM((BLK, width), jnp.float32) for _ in range(3)],
          pltpu.VMEM_SHARED((N_NODES, width), jnp.float32),
          pltpu.VMEM_SHARED((ACC_ROWS, width), jnp.float32),
          [pltpu.SemaphoreType.DMA for _ in range(3)],
      ],
  )
  def k(src_hbm, dst_hbm, feat_hbm, out_hbm, src_v, dst_v, bufs,
        feat_sh, acc, gsem):
    c = lax.axis_index("c")
    s = lax.axis_index("s")
    wid = c * NS + s
    pltpu.sync_copy(src_hbm.at[wid], src_v)
    pltpu.sync_copy(dst_hbm.at[wid], dst_v)

    frows = N_NODES // NS
    pltpu.sync_copy(feat_hbm.at[pl.ds(s * frows, frows)],
                    feat_sh.at[pl.ds(s * frows, frows)])

    def fill(i, _):
      for t in range(width // 16):
        bufs[0][i, pl.ds(t * 16, 16)] = jnp.zeros((16,), jnp.float32)
      return 0

    lax.fori_loop(0, BLK, fill, 0)
    base = s * STRIPE
    for t in range(STRIPE // BLK):
      pltpu.sync_copy(bufs[0], acc.at[pl.ds(base + t * BLK, BLK)])
    plsc.subcore_barrier()

    def gw(j, k):
      pltpu.make_async_copy(feat_sh.at[src_v.at[j]], bufs[k], gsem[k]).wait()

    def gstart(j, k):
      pltpu.async_copy(feat_sh.at[src_v.at[j]], bufs[k], gsem[k])

    def sadd(j, k):
      pltpu.sync_copy(bufs[k], acc.at[dst_v.at[j]], add=True)

    for k3 in range(3):
      gstart(k3, k3)

    def blk(t, _):
      j = 3 * t
      for k3 in range(3):
        gw(j + k3, k3)
        sadd(j + k3, k3)
        gstart(j + k3 + 3, k3)
      return 0

    lax.fori_loop(0, NBLK // 3 - 1, blk, 0)
    for jj in range(NBLK - 5, NBLK):
      gw(jj, jj % 3)
      sadd(jj, jj % 3)
      if jj + 3 < NBLK:
        gstart(jj + 3, jj % 3)
    plsc.subcore_barrier()
    for t in range(STRIPE // BLK):
      r = base + t * BLK
      pltpu.sync_copy(acc.at[pl.ds(r, BLK)], out_hbm.at[c, pl.ds(r, BLK)])

  return k(src_r, dst_r, feat)


_ROWS = 1000
_GRID = N_NODES // _ROWS


def _dinv_of(dp_ref):
  deg = dp_ref[0, :, 0:1] + dp_ref[1, :, 0:1] + 1.0
  return lax.rsqrt(deg)


def _l1_body(x_ref, w_ref, dp_ref, y_ref):
  dinv = _dinv_of(dp_ref)
  y_ref[...] = jnp.dot(
      x_ref[...], w_ref[...], preferred_element_type=jnp.float32) * dinv


def _l2_body(y_ref, p_ref, dp_ref, b1_ref, w2_ref, z_ref):
  dinv = _dinv_of(dp_ref)
  pre = (p_ref[0] + p_ref[1] + y_ref[...]) * dinv + b1_ref[...]
  h = jnp.maximum(pre, 0.0)
  z_ref[...] = jnp.dot(
      h, w2_ref[...], preferred_element_type=jnp.float32) * dinv


def _fin_body(z_ref, q_ref, dp_ref, b2_ref, o_ref):
  dinv = _dinv_of(dp_ref)
  o = (q_ref[0] + q_ref[1] + z_ref[...]) * dinv + b2_ref[...]
  col = lax.broadcasted_iota(jnp.int32, o.shape, 1)
  valid = col < NUM_CLASSES
  om = jnp.where(valid, o, -jnp.inf)
  m = jnp.max(om, axis=1, keepdims=True)
  e = jnp.where(valid, jnp.exp(om - m), 0.0)
  lse = jnp.log(jnp.sum(e, axis=1, keepdims=True)) + m
  o_ref[...] = (o - lse)[:, :NUM_CLASSES]


def _dp_spec():
  return pl.BlockSpec((NC, _ROWS, 16), lambda i: (0, i, 0))


def kernel(x, edge_index, W1, b1, W2, b2):
  src = edge_index[0].astype(jnp.int32)
  dst = edge_index[1].astype(jnp.int32)
  pad = EPAD - N_EDGES
  src_r = jnp.concatenate([src, jnp.zeros((pad,), jnp.int32)]).reshape(
      NW, NBLK, BLK)
  dst_r = jnp.concatenate([dst, jnp.full((pad,), TRASH, jnp.int32)]).reshape(
      NW, NBLK, BLK)
  w2p = jnp.pad(W2, ((0, 0), (0, CPAD - NUM_CLASSES)))
  b1r = b1.reshape(1, HIDDEN)
  b2r = jnp.pad(b2, (0, CPAD - NUM_CLASSES)).reshape(1, CPAD)

  degp = _sc_degree(dst_r)

  y = pl.pallas_call(
      _l1_body,
      grid=(_GRID,),
      in_specs=[
          pl.BlockSpec((_ROWS, D_IN), lambda i: (i, 0)),
          pl.BlockSpec((D_IN, HIDDEN), lambda i: (0, 0)),
          _dp_spec(),
      ],
      out_specs=pl.BlockSpec((_ROWS, HIDDEN), lambda i: (i, 0)),
      out_shape=jax.ShapeDtypeStruct((N_NODES, HIDDEN), jnp.float32),
  )(x, W1, degp)

  p = _sc_aggregate(src_r, dst_r, y, HIDDEN)

  z = pl.pallas_call(
      _l2_body,
      grid=(_GRID,),
      in_specs=[
          pl.BlockSpec((_ROWS, HIDDEN), lambda i: (i, 0)),
          pl.BlockSpec((NC, _ROWS, HIDDEN), lambda i: (0, i, 0)),
          _dp_spec(),
          pl.BlockSpec((1, HIDDEN), lambda i: (0, 0)),
          pl.BlockSpec((HIDDEN, CPAD), lambda i: (0, 0)),
      ],
      out_specs=pl.BlockSpec((_ROWS, CPAD), lambda i: (i, 0)),
      out_shape=jax.ShapeDtypeStruct((N_NODES, CPAD), jnp.float32),
  )(y, p, degp, b1r, w2p)

  q = _sc_aggregate(src_r, dst_r, z, CPAD)

  out = pl.pallas_call(
      _fin_body,
      grid=(_GRID,),
      in_specs=[
          pl.BlockSpec((_ROWS, CPAD), lambda i: (i, 0)),
          pl.BlockSpec((NC, _ROWS, CPAD), lambda i: (0, i, 0)),
          _dp_spec(),
          pl.BlockSpec((1, CPAD), lambda i: (0, 0)),
      ],
      out_specs=pl.BlockSpec((_ROWS, NUM_CLASSES), lambda i: (i, 0)),
      out_shape=jax.ShapeDtypeStruct((N_NODES, NUM_CLASSES), jnp.float32),
  )(z, q, degp, b2r)

  return out


# speedup vs baseline: 36.3215x; 1.0029x over previous
import functools

import jax
import jax.numpy as jnp
from jax import lax
from jax.experimental import pallas as pl
from jax.experimental.pallas import tpu as pltpu
from jax.experimental.pallas import tpu_sc as plsc

N_NODES = 10000
N_EDGES = 320000
D_IN = 128
HIDDEN = 64
NUM_CLASSES = 40
CPAD = 48

NC = 2
NS = 16
NW = NC * NS
BLK = 128
NBLK = 80
EPAD = NW * NBLK * BLK
ACC_ROWS = 10240
STRIPE = ACC_ROWS // NS
TRASH = N_NODES + 16

_mesh = plsc.VectorSubcoreMesh(core_axis_name="c", subcore_axis_name="s")
_sc_params = pltpu.CompilerParams(use_tc_tiling_on_sc=False)


def _sc_degree(dst_r):
  @functools.partial(
      pl.kernel,
      mesh=_mesh,
      out_type=jax.ShapeDtypeStruct((NC, ACC_ROWS, 16), jnp.float32),
      compiler_params=_sc_params,
      scratch_types=[
          pltpu.VMEM((NBLK, BLK), jnp.int32),
          pltpu.VMEM((BLK, 16), jnp.float32),
          pltpu.VMEM((BLK, 16), jnp.float32),
          pltpu.VMEM_SHARED((ACC_ROWS, 16), jnp.float32),
          pltpu.SemaphoreType.DMA,
      ],
  )
  def k(dst_hbm, out_hbm, dst_v, ones_v, zeros_v, acc, sem):
    c = lax.axis_index("c")
    s = lax.axis_index("s")
    wid = c * NS + s
    pltpu.sync_copy(dst_hbm.at[wid], dst_v)

    def fill(i, _):
      ones_v[i, :] = jnp.full((16,), 1.0, jnp.float32)
      zeros_v[i, :] = jnp.zeros((16,), jnp.float32)
      return 0

    lax.fori_loop(0, BLK, fill, 0)
    base = s * STRIPE
    for t in range(STRIPE // BLK):
      pltpu.sync_copy(zeros_v, acc.at[pl.ds(base + t * BLK, BLK)])
    plsc.subcore_barrier()

    def blk(j, _):
      pltpu.sync_copy(ones_v, acc.at[dst_v.at[j]], add=True)
      return 0

    lax.fori_loop(0, NBLK, blk, 0)
    plsc.subcore_barrier()
    for t in range(STRIPE // BLK):
      r = base + t * BLK
      pltpu.sync_copy(acc.at[pl.ds(r, BLK)], out_hbm.at[c, pl.ds(r, BLK)])

  return k(dst_r)


def _sc_aggregate(src_r, dst_r, feat, width):
  @functools.partial(
      pl.kernel,
      mesh=_mesh,
      out_type=jax.ShapeDtypeStruct((NC, ACC_ROWS, width), jnp.float32),
      compiler_params=_sc_params,
      scratch_types=[
          pltpu.VMEM((NBLK, BLK), jnp.int32),
          pltpu.VMEM((NBLK, BLK), jnp.int32),
          [pltpu.VMEM((BLK, width), jnp.float32) for _ in range(3)],
          pltpu.VMEM_SHARED((N_NODES, width), jnp.float32),
          pltpu.VMEM_SHARED((ACC_ROWS, width), jnp.float32),
          [pltpu.SemaphoreType.DMA for _ in range(3)],
      ],
  )
  def k(src_hbm, dst_hbm, feat_hbm, out_hbm, src_v, dst_v, bufs,
        feat_sh, acc, gsem):
    c = lax.axis_index("c")
    s = lax.axis_index("s")
    wid = c * NS + s
    pltpu.sync_copy(src_hbm.at[wid], src_v)
    pltpu.sync_copy(dst_hbm.at[wid], dst_v)

    frows = N_NODES // NS
    pltpu.sync_copy(feat_hbm.at[pl.ds(s * frows, frows)],
                    feat_sh.at[pl.ds(s * frows, frows)])

    def fill(i, _):
      for t in range(width // 16):
        bufs[0][i, pl.ds(t * 16, 16)] = jnp.zeros((16,), jnp.float32)
      return 0

    lax.fori_loop(0, BLK, fill, 0)
    base = s * STRIPE
    for t in range(STRIPE // BLK):
      pltpu.sync_copy(bufs[0], acc.at[pl.ds(base + t * BLK, BLK)])
    plsc.subcore_barrier()

    def gw(j, k):
      pltpu.make_async_copy(feat_sh.at[src_v.at[j]], bufs[k], gsem[k]).wait()

    def gstart(j, k):
      pltpu.async_copy(feat_sh.at[src_v.at[j]], bufs[k], gsem[k])

    def sadd(j, k):
      pltpu.sync_copy(bufs[k], acc.at[dst_v.at[j]], add=True)

    for k3 in range(3):
      gstart(k3, k3)

    def blk(t, _):
      j = 3 * t
      for k3 in range(3):
        gw(j + k3, k3)
        sadd(j + k3, k3)
        gstart(j + k3 + 3, k3)
      return 0

    lax.fori_loop(0, NBLK // 3 - 1, blk, 0)
    for jj in range(NBLK - 5, NBLK):
      gw(jj, jj % 3)
      sadd(jj, jj % 3)
      if jj + 3 < NBLK:
        gstart(jj + 3, jj % 3)
    plsc.subcore_barrier()
    for t in range(STRIPE // BLK):
      r = base + t * BLK
      pltpu.sync_copy(acc.at[pl.ds(r, BLK)], out_hbm.at[c, pl.ds(r, BLK)])

  return k(src_r, dst_r, feat)


_ROWS = 1000
_GRID = N_NODES // _ROWS


def _dinv_of(dp_ref):
  deg = dp_ref[0, :, 0:1] + dp_ref[1, :, 0:1] + 1.0
  return lax.rsqrt(deg)


def _xw_body(x_ref, w_ref, xw_ref):
  xw_ref[...] = jnp.dot(
      x_ref[...], w_ref[...], preferred_element_type=jnp.float32)


def _scale_body(xw_ref, dp_ref, y_ref):
  y_ref[...] = xw_ref[...] * _dinv_of(dp_ref)


def _l2_body(y_ref, p_ref, dp_ref, b1_ref, w2_ref, z_ref):
  dinv = _dinv_of(dp_ref)
  pre = (p_ref[0] + p_ref[1] + y_ref[...]) * dinv + b1_ref[...]
  h = jnp.maximum(pre, 0.0)
  z_ref[...] = jnp.dot(
      h, w2_ref[...], preferred_element_type=jnp.float32) * dinv


def _fin_body(z_ref, q_ref, dp_ref, b2_ref, o_ref):
  dinv = _dinv_of(dp_ref)
  o = (q_ref[0] + q_ref[1] + z_ref[...]) * dinv + b2_ref[...]
  col = lax.broadcasted_iota(jnp.int32, o.shape, 1)
  valid = col < NUM_CLASSES
  om = jnp.where(valid, o, -jnp.inf)
  m = jnp.max(om, axis=1, keepdims=True)
  e = jnp.where(valid, jnp.exp(om - m), 0.0)
  lse = jnp.log(jnp.sum(e, axis=1, keepdims=True)) + m
  o_ref[...] = (o - lse)[:, :NUM_CLASSES]


def _dp_spec():
  return pl.BlockSpec((NC, _ROWS, 16), lambda i: (0, i, 0))


def kernel(x, edge_index, W1, b1, W2, b2):
  src = edge_index[0].astype(jnp.int32)
  dst = edge_index[1].astype(jnp.int32)
  pad = EPAD - N_EDGES
  src_r = jnp.concatenate([src, jnp.zeros((pad,), jnp.int32)]).reshape(
      NW, NBLK, BLK)
  dst_r = jnp.concatenate([dst, jnp.full((pad,), TRASH, jnp.int32)]).reshape(
      NW, NBLK, BLK)
  w2p = jnp.pad(W2, ((0, 0), (0, CPAD - NUM_CLASSES)))
  b1r = b1.reshape(1, HIDDEN)
  b2r = jnp.pad(b2, (0, CPAD - NUM_CLASSES)).reshape(1, CPAD)

  degp = _sc_degree(dst_r)

  xw = pl.pallas_call(
      _xw_body,
      grid=(_GRID,),
      in_specs=[
          pl.BlockSpec((_ROWS, D_IN), lambda i: (i, 0)),
          pl.BlockSpec((D_IN, HIDDEN), lambda i: (0, 0)),
      ],
      out_specs=pl.BlockSpec((_ROWS, HIDDEN), lambda i: (i, 0)),
      out_shape=jax.ShapeDtypeStruct((N_NODES, HIDDEN), jnp.float32),
  )(x, W1)

  y = pl.pallas_call(
      _scale_body,
      grid=(_GRID,),
      in_specs=[
          pl.BlockSpec((_ROWS, HIDDEN), lambda i: (i, 0)),
          _dp_spec(),
      ],
      out_specs=pl.BlockSpec((_ROWS, HIDDEN), lambda i: (i, 0)),
      out_shape=jax.ShapeDtypeStruct((N_NODES, HIDDEN), jnp.float32),
  )(xw, degp)

  p = _sc_aggregate(src_r, dst_r, y, HIDDEN)

  z = pl.pallas_call(
      _l2_body,
      grid=(_GRID,),
      in_specs=[
          pl.BlockSpec((_ROWS, HIDDEN), lambda i: (i, 0)),
          pl.BlockSpec((NC, _ROWS, HIDDEN), lambda i: (0, i, 0)),
          _dp_spec(),
          pl.BlockSpec((1, HIDDEN), lambda i: (0, 0)),
          pl.BlockSpec((HIDDEN, CPAD), lambda i: (0, 0)),
      ],
      out_specs=pl.BlockSpec((_ROWS, CPAD), lambda i: (i, 0)),
      out_shape=jax.ShapeDtypeStruct((N_NODES, CPAD), jnp.float32),
  )(y, p, degp, b1r, w2p)

  q = _sc_aggregate(src_r, dst_r, z, CPAD)

  out = pl.pallas_call(
      _fin_body,
      grid=(_GRID,),
      in_specs=[
          pl.BlockSpec((_ROWS, CPAD), lambda i: (i, 0)),
          pl.BlockSpec((NC, _ROWS, CPAD), lambda i: (0, i, 0)),
          _dp_spec(),
          pl.BlockSpec((1, CPAD), lambda i: (0, 0)),
      ],
      out_specs=pl.BlockSpec((_ROWS, NUM_CLASSES), lambda i: (i, 0)),
      out_shape=jax.ShapeDtypeStruct((N_NODES, NUM_CLASSES), jnp.float32),
  )(z, q, degp, b2r)

  return out
